# baseline (speedup 1.0000x reference)
import functools
import jax
import jax.numpy as jnp
from jax import lax
from jax.experimental import pallas as pl
from jax.experimental.pallas import tpu as pltpu, tpu_sc as plsc

_NW = 16


def _make_sc_kernel(V, F):
    NW = _NW
    RPW = (V + NW - 1) // NW
    FPW = (F + NW - 1) // NW
    WINF = 1024
    NWINF = (FPW + WINF - 1) // WINF
    CW = WINF * 3
    KEYSLOTS = NW * NWINF * WINF * 6
    MAILCAP = KEYSLOTS + 17 * 128 + CW
    SORTCAP = 32768
    HRPW = (RPW + 1) // 2
    NBV = ((RPW + 1 + 15) // 16) * 16
    ACCR = ((V + 16) + 15) // 16 * 16

    mesh = plsc.VectorSubcoreMesh(
        core_axis_name="c", subcore_axis_name="s", num_cores=1
    )

    @functools.partial(
        pl.kernel,
        out_type=[
            jax.ShapeDtypeStruct((V, 8), jnp.float32),
            jax.ShapeDtypeStruct((MAILCAP,), jnp.int32),
        ],
        mesh=mesh,
        compiler_params=pltpu.CompilerParams(
            needs_layout_passes=False, use_tc_tiling_on_sc=False
        ),
        scratch_types=[
            pltpu.VMEM((CW,), jnp.int32),
            pltpu.VMEM((32,), jnp.int32),
            pltpu.VMEM((512,), jnp.int32),
            pltpu.VMEM((32,), jnp.int32),
            pltpu.VMEM((3, 128), jnp.int32),
            pltpu.VMEM((3, 128), jnp.int32),
            pltpu.VMEM((NBV,), jnp.int32),
            pltpu.VMEM((SORTCAP,), jnp.int32),
            pltpu.VMEM((V,), jnp.int32),
            pltpu.VMEM((1, 128), jnp.int32),
            pltpu.VMEM((1, 128), jnp.int32),
            pltpu.VMEM((128, 8), jnp.float32),
            pltpu.VMEM_SHARED((ACCR, 8), jnp.float32),
            pltpu.VMEM_SHARED((512,), jnp.int32),
        ],
    )
    def sc_kernel(faces_hbm, verts_hbm, acc_out, mail_hbm, win, histd, histall,
                  cur, sstg, kstg, rowhist, sorted_b, stamp, ridx, cidx,
                  rows128, acc_sh, hist_sh):
        sid = lax.axis_index("s")
        cid = lax.axis_index("c")
        wid = sid + 16 * cid
        iota = lax.iota(jnp.int32, 16)
        i3 = iota * 3

        def face_keys(k, winidx):
            base = k * 48
            a = plsc.load_gather(win, [i3 + base])
            b = plsc.load_gather(win, [i3 + base + 1])
            c = plsc.load_gather(win, [i3 + base + 2])
            gface = winidx * WINF + k * 16 + iota
            valid = gface < FPW

            def pack(u, v):
                return jnp.where(valid, (u << 16) | v, -1)

            return [pack(b, c), pack(c, b), pack(c, a),
                    pack(a, c), pack(a, b), pack(b, a)]

        def dest_of(kv):
            r = lax.shift_right_logical(kv, 16)
            return jnp.minimum(r // RPW, 16)

        histd[pl.ds(0, 16)] = jnp.zeros((16,), jnp.int32)
        histd[pl.ds(16, 16)] = jnp.zeros((16,), jnp.int32)

        def phA_win(winidx, _):
            pltpu.sync_copy(faces_hbm.at[wid, pl.ds(pl.multiple_of(winidx * CW, CW), CW)], win)

            def phA_k(k, _):
                for kv in face_keys(k, winidx):
                    d = dest_of(kv)
                    cnt, lastm = plsc.scan_count(d)
                    plsc.addupdate_scatter(histd, [d], cnt, mask=lastm)
                return 0

            lax.fori_loop(0, WINF // 16, phA_k, 0)
            return 0

        lax.fori_loop(0, NWINF, phA_win, 0)
        pltpu.sync_copy(histd, hist_sh.at[pl.ds(pl.multiple_of(wid * 32, 32), 32)])
        plsc.subcore_barrier()

        pltpu.sync_copy(hist_sh, histall)
        dstbase = jnp.int32(0)
        seg_lo = jnp.int32(0)
        t_w = jnp.int32(0)
        for d in range(17):
            v = plsc.load_gather(histall, [iota * 32 + d])
            tot = jnp.sum(v)
            offw = plsc.cumsum(v) - v + dstbase
            plsc.store_scatter(
                cur, [jnp.full((16,), d, jnp.int32)], offw, mask=(iota == wid)
            )
            seg_lo = jnp.where(wid == d, dstbase, seg_lo)
            t_w = jnp.where(wid == d, tot, t_w)
            dstbase = dstbase + jnp.bitwise_and(tot + 127, -128)

        def phB_win(winidx, _):
            pltpu.sync_copy(faces_hbm.at[wid, pl.ds(pl.multiple_of(winidx * CW, CW), CW)], win)

            def phB_g(g, _):
                for q in range(4):
                    keys6 = face_keys(g * 4 + q, winidx)
                    for j, kv in enumerate(keys6):
                        d = dest_of(kv)
                        cnt, lastm = plsc.scan_count(d)
                        basev = plsc.load_gather(cur, [d])
                        slot = basev + cnt - 1
                        plsc.addupdate_scatter(cur, [d], cnt, mask=lastm)
                        t = q * 6 + j
                        sstg[t // 8, pl.ds((t % 8) * 16, 16)] = slot
                        kstg[t // 8, pl.ds((t % 8) * 16, 16)] = kv
                for rr in range(3):
                    pltpu.sync_copy(kstg.at[rr], mail_hbm.at[sstg.at[rr]])
                return 0

            lax.fori_loop(0, WINF // 64, phB_g, 0)
            return 0

        lax.fori_loop(0, NWINF, phB_win, 0)
        plsc.subcore_barrier()

        def zrh(i, _):
            rowhist[pl.ds(i * 16, 16)] = jnp.zeros((16,), jnp.int32)
            return 0

        lax.fori_loop(0, NBV // 16, zrh, 0)

        nwc = (t_w + CW - 1) // CW

        def localrow(kv, valid):
            lr = lax.shift_right_logical(kv, 16) - wid * RPW
            return jnp.where(valid, lr, RPW)

        def phC1_win(winidx, _):
            pltpu.sync_copy(mail_hbm.at[pl.ds(pl.multiple_of(seg_lo + winidx * CW, 128), CW)], win)

            def phC1_v(j, _):
                kv = win[pl.ds(j * 16, 16)]
                gi = winidx * CW + j * 16 + iota
                lr = localrow(kv, gi < t_w)
                cnt, lastm = plsc.scan_count(lr)
                plsc.addupdate_scatter(rowhist, [lr], cnt, mask=lastm)
                return 0

            lax.fori_loop(0, CW // 16, phC1_v, 0)
            return 0

        lax.fori_loop(0, nwc, phC1_win, 0)

        carry = jnp.int32(0)
        for i in range(NBV // 16):
            v = rowhist[pl.ds(i * 16, 16)]
            rowhist[pl.ds(i * 16, 16)] = plsc.cumsum(v) - v + carry
            carry = carry + jnp.sum(v)

        hb1 = jnp.max(
            plsc.load_gather(rowhist, [jnp.full((16,), HRPW, jnp.int32)])
        )
        hb2 = jnp.max(
            plsc.load_gather(rowhist, [jnp.full((16,), RPW, jnp.int32)])
        )

        def zst(i, _):
            stamp[pl.ds(i * 16, 16)] = jnp.full((16,), -1, jnp.int32)
            return 0

        lax.fori_loop(0, (V + 15) // 16, zst, 0)

        for k in range(8):
            for j in range(8):
                plsc.store_scatter(
                    rows128,
                    [iota + k * 16, jnp.full((16,), j, jnp.int32)],
                    jnp.zeros((16,), jnp.float32),
                )

        def zacc(i, _):
            pltpu.sync_copy(
                rows128.at[pl.ds(0, 125)],
                acc_sh.at[pl.ds(wid * RPW + i * 125, 125)],
            )
            return 0

        lax.fori_loop(0, RPW // 125, zacc, 0)

        for h in range(2):
            halfbase = jnp.where(h == 0, jnp.int32(0), hb1)
            halfcount = jnp.where(h == 0, hb1, hb2 - hb1)
            lo = h * HRPW
            hi = jnp.minimum((h + 1) * HRPW, RPW)

            def phC3_win(winidx, _):
                pltpu.sync_copy(
                    mail_hbm.at[pl.ds(pl.multiple_of(seg_lo + winidx * CW, 128), CW)],
                    win,
                )

                def phC3_v(j, _):
                    kv = win[pl.ds(j * 16, 16)]
                    gi = winidx * CW + j * 16 + iota
                    lr = localrow(kv, gi < t_w)
                    inhalf = jnp.logical_and(lr >= lo, lr < hi)
                    cnt, lastm = plsc.scan_count(lr)
                    basev = plsc.load_gather(rowhist, [lr])
                    slot = basev + cnt - 1 - halfbase
                    plsc.addupdate_scatter(
                        rowhist, [lr], cnt, mask=jnp.logical_and(lastm, inhalf)
                    )
                    relslot = jnp.where(inhalf, slot, SORTCAP - 16 + iota)
                    plsc.store_scatter(sorted_b, [relslot], kv)
                    return 0

                lax.fori_loop(0, CW // 16, phC3_v, 0)
                return 0

            lax.fori_loop(0, nwc, phC3_win, 0)

            ng = (halfcount + 127) // 128

            def phC4_g(g, _):
                for k in range(8):
                    kv = sorted_b[pl.ds(g * 128 + k * 16, 16)]
                    gi = g * 128 + k * 16 + iota
                    valid = gi < halfcount
                    r = lax.shift_right_logical(kv, 16)
                    c2 = jnp.minimum(jnp.bitwise_and(kv, 0xFFFF), V - 1)
                    cntk, _lk = plsc.scan_count(kv, mask=valid)
                    first = jnp.logical_and(cntk == 1, valid)
                    stv = plsc.load_gather(stamp, [c2])
                    emit = jnp.logical_and(first, stv != r)
                    cntc, lastc = plsc.scan_count(c2, mask=valid)
                    plsc.store_scatter(
                        stamp, [c2], r, mask=jnp.logical_and(lastc, valid)
                    )
                    ridx[0, pl.ds(k * 16, 16)] = jnp.where(emit, r, V)
                    cidx[0, pl.ds(k * 16, 16)] = jnp.where(emit, c2, 0)
                pltpu.sync_copy(verts_hbm.at[cidx.at[0]], rows128)
                pltpu.sync_copy(rows128, acc_sh.at[ridx.at[0]], add=True)
                return 0

            lax.fori_loop(0, ng, phC4_g, 0)

        pltpu.sync_copy(
            acc_sh.at[pl.ds(wid * RPW, RPW)], acc_out.at[pl.ds(wid * RPW, RPW)]
        )

    return sc_kernel


def _tc_body(a_ref, v_ref, out_ref):
    a0 = a_ref[0:1, :]
    a1 = a_ref[1:2, :]
    a2 = a_ref[2:3, :]
    a3 = a_ref[3:4, :]
    lx = a3 * v_ref[0:1, :] - a0
    ly = a3 * v_ref[1:2, :] - a1
    lz = a3 * v_ref[2:3, :] - a2
    out_ref[0, 0] = jnp.sum(jnp.sqrt(lx * lx + ly * ly + lz * lz))


def kernel(verts, faces):
    V = verts.shape[0]
    F = faces.shape[0]
    NW = _NW
    FPW = (F + NW - 1) // NW
    WINF = 1024
    NWINF = (FPW + WINF - 1) // WINF
    CW = WINF * 3

    ff = faces.reshape(NW, (F // NW) * 3)
    faces_pad = jnp.pad(ff, ((0, 0), (0, NWINF * CW - ff.shape[1])))
    verts8 = jnp.concatenate(
        [verts, jnp.ones((V, 1), jnp.float32), jnp.zeros((V, 4), jnp.float32)],
        axis=1,
    )
    acc, _ = _make_sc_kernel(V, F)(faces_pad, verts8)

    pad = (-V) % 128
    aT = jnp.pad(acc[:, :4].T, ((0, 4), (0, pad)))
    vT = jnp.pad(verts.T, ((0, 5), (0, pad)))
    total = pl.pallas_call(
        _tc_body,
        out_shape=jax.ShapeDtypeStruct((1, 1), jnp.float32),
        out_specs=pl.BlockSpec(memory_space=pltpu.SMEM),
    )(aT, vT)
    return total[0, 0] / V


# speedup vs baseline: 1.0763x; 1.0763x over previous
import functools
import jax
import jax.numpy as jnp
from jax import lax
from jax.experimental import pallas as pl
from jax.experimental.pallas import tpu as pltpu, tpu_sc as plsc

_NW = 16


def _make_sc_kernel(V, F):
    NW = _NW
    RPW = (V + NW - 1) // NW
    FPW = (F + NW - 1) // NW
    WINF = 1024
    NWINF = (FPW + WINF - 1) // WINF
    CW = WINF * 3
    KEYSLOTS = NW * NWINF * WINF * 6
    MAILCAP = KEYSLOTS + 17 * 128 + CW
    SORTCAP = 32768
    HRPW = (RPW + 1) // 2
    NBV = ((RPW + 1 + 15) // 16) * 16
    ACCR = ((V + 16) + 15) // 16 * 16

    mesh = plsc.VectorSubcoreMesh(
        core_axis_name="c", subcore_axis_name="s", num_cores=1
    )

    @functools.partial(
        pl.kernel,
        out_type=[
            jax.ShapeDtypeStruct((V, 8), jnp.float32),
            jax.ShapeDtypeStruct((MAILCAP,), jnp.int32),
        ],
        mesh=mesh,
        compiler_params=pltpu.CompilerParams(
            needs_layout_passes=False, use_tc_tiling_on_sc=False,
            has_side_effects=True
        ),
        scratch_types=[
            pltpu.VMEM((CW,), jnp.int32),
            pltpu.VMEM((32,), jnp.int32),
            pltpu.VMEM((512,), jnp.int32),
            pltpu.VMEM((32,), jnp.int32),
            pltpu.VMEM((3, 128), jnp.int32),
            pltpu.VMEM((3, 128), jnp.int32),
            pltpu.VMEM((NBV,), jnp.int32),
            pltpu.VMEM((SORTCAP,), jnp.int32),
            pltpu.VMEM((V,), jnp.int32),
            pltpu.VMEM((8, 128), jnp.int32),
            pltpu.VMEM((1024,), jnp.int32),
            pltpu.VMEM((1024, 8), jnp.float32),
            pltpu.VMEM_SHARED((ACCR, 8), jnp.float32),
            pltpu.VMEM_SHARED((512,), jnp.int32),
        ],
    )
    def sc_kernel(faces_hbm, verts_hbm, acc_out, mail_hbm, win, histd, histall,
                  cur, sstg, kstg, rowhist, sorted_b, stamp, ridx, cidx,
                  rows1k, acc_sh, hist_sh):
        sid = lax.axis_index("s")
        cid = lax.axis_index("c")
        wid = sid + 16 * cid
        iota = lax.iota(jnp.int32, 16)
        i3 = iota * 3

        def face_keys(k, winidx):
            base = k * 48
            a = plsc.load_gather(win, [i3 + base])
            b = plsc.load_gather(win, [i3 + base + 1])
            c = plsc.load_gather(win, [i3 + base + 2])
            gface = winidx * WINF + k * 16 + iota
            valid = gface < FPW

            def pack(u, v):
                return jnp.where(valid, (u << 16) | v, -1)

            return [pack(b, c), pack(c, b), pack(c, a),
                    pack(a, c), pack(a, b), pack(b, a)]

        def dest_of(kv):
            r = lax.shift_right_logical(kv, 16)
            return jnp.minimum(r // RPW, 16)

        histd[pl.ds(0, 16)] = jnp.zeros((16,), jnp.int32)
        histd[pl.ds(16, 16)] = jnp.zeros((16,), jnp.int32)

        def phA_win(winidx, _):
            pltpu.sync_copy(faces_hbm.at[wid, pl.ds(pl.multiple_of(winidx * CW, CW), CW)], win)

            def phA_k(k, _):
                for kv in face_keys(k, winidx):
                    d = dest_of(kv)
                    cnt, lastm = plsc.scan_count(d)
                    plsc.addupdate_scatter(histd, [d], cnt, mask=lastm)
                return 0

            lax.fori_loop(0, WINF // 16, phA_k, 0)
            return 0

        lax.fori_loop(0, NWINF, phA_win, 0)
        pltpu.sync_copy(histd, hist_sh.at[pl.ds(pl.multiple_of(wid * 32, 32), 32)])
        plsc.subcore_barrier()

        pltpu.sync_copy(hist_sh, histall)
        dstbase = jnp.int32(0)
        seg_lo = jnp.int32(0)
        t_w = jnp.int32(0)
        for d in range(17):
            v = plsc.load_gather(histall, [iota * 32 + d])
            tot = jnp.sum(v)
            offw = plsc.cumsum(v) - v + dstbase
            plsc.store_scatter(
                cur, [jnp.full((16,), d, jnp.int32)], offw, mask=(iota == wid)
            )
            seg_lo = jnp.where(wid == d, dstbase, seg_lo)
            t_w = jnp.where(wid == d, tot, t_w)
            dstbase = dstbase + jnp.bitwise_and(tot + 127, -128)

        def phB_win(winidx, _):
            pltpu.sync_copy(faces_hbm.at[wid, pl.ds(pl.multiple_of(winidx * CW, CW), CW)], win)

            def phB_g(g, _):
                for q in range(4):
                    keys6 = face_keys(g * 4 + q, winidx)
                    for j, kv in enumerate(keys6):
                        d = dest_of(kv)
                        cnt, lastm = plsc.scan_count(d)
                        basev = plsc.load_gather(cur, [d])
                        slot = basev + cnt - 1
                        plsc.addupdate_scatter(cur, [d], cnt, mask=lastm)
                        t = q * 6 + j
                        sstg[t // 8, pl.ds((t % 8) * 16, 16)] = slot
                        kstg[t // 8, pl.ds((t % 8) * 16, 16)] = kv
                for rr in range(3):
                    pltpu.sync_copy(kstg.at[rr], mail_hbm.at[sstg.at[rr]])
                return 0

            lax.fori_loop(0, WINF // 64, phB_g, 0)
            return 0

        lax.fori_loop(0, NWINF, phB_win, 0)
        plsc.subcore_barrier()

        def zrh(i, _):
            rowhist[pl.ds(i * 16, 16)] = jnp.zeros((16,), jnp.int32)
            return 0

        lax.fori_loop(0, NBV // 16, zrh, 0)

        nwc = (t_w + CW - 1) // CW

        def localrow(kv, valid):
            lr = lax.shift_right_logical(kv, 16) - wid * RPW
            return jnp.where(valid, lr, RPW)

        def phC1_win(winidx, _):
            pltpu.sync_copy(mail_hbm.at[pl.ds(pl.multiple_of(seg_lo + winidx * CW, 128), CW)], win)

            def phC1_v(j, _):
                kv = win[pl.ds(j * 16, 16)]
                gi = winidx * CW + j * 16 + iota
                lr = localrow(kv, gi < t_w)
                cnt, lastm = plsc.scan_count(lr)
                plsc.addupdate_scatter(rowhist, [lr], cnt, mask=lastm)
                return 0

            lax.fori_loop(0, CW // 16, phC1_v, 0)
            return 0

        lax.fori_loop(0, nwc, phC1_win, 0)

        carry = jnp.int32(0)
        for i in range(NBV // 16):
            v = rowhist[pl.ds(i * 16, 16)]
            rowhist[pl.ds(i * 16, 16)] = plsc.cumsum(v) - v + carry
            carry = carry + jnp.sum(v)

        hb1 = jnp.max(
            plsc.load_gather(rowhist, [jnp.full((16,), HRPW, jnp.int32)])
        )
        hb2 = jnp.max(
            plsc.load_gather(rowhist, [jnp.full((16,), RPW, jnp.int32)])
        )

        def zst(i, _):
            stamp[pl.ds(i * 16, 16)] = jnp.full((16,), -1, jnp.int32)
            return 0

        lax.fori_loop(0, (V + 15) // 16, zst, 0)

        def zrow(i, _):
            for j in range(8):
                plsc.store_scatter(
                    rows1k,
                    [iota + i * 16, jnp.full((16,), j, jnp.int32)],
                    jnp.zeros((16,), jnp.float32),
                )
            return 0

        lax.fori_loop(0, 64, zrow, 0)

        def zacc(i, _):
            pltpu.sync_copy(
                rows1k.at[pl.ds(0, 625)],
                acc_sh.at[pl.ds(wid * RPW + i * 625, 625)],
            )
            return 0

        lax.fori_loop(0, RPW // 625, zacc, 0)

        for h in range(2):
            halfbase = jnp.where(h == 0, jnp.int32(0), hb1)
            halfcount = jnp.where(h == 0, hb1, hb2 - hb1)
            lo = h * HRPW
            hi = jnp.minimum((h + 1) * HRPW, RPW)

            def phC3_win(winidx, _):
                pltpu.sync_copy(
                    mail_hbm.at[pl.ds(pl.multiple_of(seg_lo + winidx * CW, 128), CW)],
                    win,
                )

                def phC3_v(j, _):
                    kv = win[pl.ds(j * 16, 16)]
                    gi = winidx * CW + j * 16 + iota
                    lr = localrow(kv, gi < t_w)
                    inhalf = jnp.logical_and(lr >= lo, lr < hi)
                    cnt, lastm = plsc.scan_count(lr)
                    basev = plsc.load_gather(rowhist, [lr])
                    slot = basev + cnt - 1 - halfbase
                    plsc.addupdate_scatter(
                        rowhist, [lr], cnt, mask=jnp.logical_and(lastm, inhalf)
                    )
                    relslot = jnp.where(inhalf, slot, SORTCAP - 16 + iota)
                    plsc.store_scatter(sorted_b, [relslot], kv)
                    return 0

                lax.fori_loop(0, CW // 16, phC3_v, 0)
                return 0

            lax.fori_loop(0, nwc, phC3_win, 0)

            ng = (halfcount + 1023) // 1024

            def phC4_g(g, _):
                for k in range(64):
                    kv = sorted_b[pl.ds(g * 1024 + k * 16, 16)]
                    gi = g * 1024 + k * 16 + iota
                    valid = gi < halfcount
                    r = lax.shift_right_logical(kv, 16)
                    c2 = jnp.minimum(jnp.bitwise_and(kv, 0xFFFF), V - 1)
                    cntk, _lk = plsc.scan_count(kv, mask=valid)
                    first = jnp.logical_and(cntk == 1, valid)
                    stv = plsc.load_gather(stamp, [c2])
                    emit = jnp.logical_and(first, stv != r)
                    cntc, lastc = plsc.scan_count(c2, mask=valid)
                    plsc.store_scatter(
                        stamp, [c2], r, mask=jnp.logical_and(lastc, valid)
                    )
                    ridx[k // 8, pl.ds((k % 8) * 16, 16)] = jnp.where(emit, r, V)
                    cidx[pl.ds(k * 16, 16)] = jnp.where(emit, c2, 0)
                pltpu.sync_copy(verts_hbm.at[cidx], rows1k)
                for rr in range(8):
                    pltpu.sync_copy(
                        rows1k.at[pl.ds(rr * 128, 128)],
                        acc_sh.at[ridx.at[rr]],
                        add=True,
                    )
                return 0

            lax.fori_loop(0, ng, phC4_g, 0)

        pltpu.sync_copy(
            acc_sh.at[pl.ds(wid * RPW, RPW)], acc_out.at[pl.ds(wid * RPW, RPW)]
        )

    return sc_kernel


def _tc_body(a_ref, v_ref, out_ref):
    a0 = a_ref[0:1, :]
    a1 = a_ref[1:2, :]
    a2 = a_ref[2:3, :]
    a3 = a_ref[3:4, :]
    lx = a3 * v_ref[0:1, :] - a0
    ly = a3 * v_ref[1:2, :] - a1
    lz = a3 * v_ref[2:3, :] - a2
    out_ref[0, 0] = jnp.sum(jnp.sqrt(lx * lx + ly * ly + lz * lz))


def kernel(verts, faces):
    V = verts.shape[0]
    F = faces.shape[0]
    NW = _NW
    FPW = (F + NW - 1) // NW
    WINF = 1024
    NWINF = (FPW + WINF - 1) // WINF
    CW = WINF * 3

    ff = faces.reshape(NW, (F // NW) * 3)
    faces_pad = jnp.pad(ff, ((0, 0), (0, NWINF * CW - ff.shape[1])))
    verts8 = jnp.concatenate(
        [verts, jnp.ones((V, 1), jnp.float32), jnp.zeros((V, 4), jnp.float32)],
        axis=1,
    )
    acc, _ = _make_sc_kernel(V, F)(faces_pad, verts8)

    pad = (-V) % 128
    aT = jnp.pad(acc[:, :4].T, ((0, 4), (0, pad)))
    vT = jnp.pad(verts.T, ((0, 5), (0, pad)))
    total = pl.pallas_call(
        _tc_body,
        out_shape=jax.ShapeDtypeStruct((1, 1), jnp.float32),
        out_specs=pl.BlockSpec(memory_space=pltpu.SMEM),
    )(aT, vT)
    return total[0, 0] / V
